# baseline (speedup 1.0000x reference)
import jax
import jax.numpy as jnp
from jax.experimental import pallas as pl


def _copy_body(emb_ref, out_ref):
    out_ref[...] = emb_ref[...][None]


def kernel(x, emb):
    seq_len = x.shape[1]
    emb_dim = emb.shape[1]
    blk = 512
    out = pl.pallas_call(
        _copy_body,
        grid=(seq_len // blk,),
        in_specs=[pl.BlockSpec((blk, emb_dim), lambda i: (i, 0))],
        out_specs=pl.BlockSpec((1, blk, emb_dim), lambda i: (0, i, 0)),
        out_shape=jax.ShapeDtypeStruct((1, seq_len, emb_dim), emb.dtype),
    )(emb)
    return out
